# baseline (speedup 1.0000x reference)
import functools

import jax
import jax.numpy as jnp
from jax import lax
from jax.experimental import pallas as pl
from jax.experimental.pallas import tpu as pltpu
from jax.experimental.pallas import tpu_sc as plsc

N = 10000
E = 320000
D_FEAT = 128
D_EDGE = 16
NTYPE_DIM = 64
ETYPE_DIM = 16

NC = 2
NS = 16
NW = NC * NS

NODE_PW = 312
NODE_TAIL = N - NW * NODE_PW
NODE_G = 104

EDGE_PW = E // NW
EC = 400
NB = EDGE_PW // EC
EG = 80


def _body(x, eattr, ntypes, etypes, ntab, etab, xcat, ecat,
          nidx_v, nemb_v, nx_v, eidx_v, eemb_v, efeat_v, sem):
    wid = lax.axis_index("s") * NC + lax.axis_index("c")

    nbase = pl.multiple_of(wid * NODE_PW, 8)
    pltpu.sync_copy(ntypes.at[pl.ds(nbase, NODE_PW)], nidx_v)
    for j in range(NODE_PW // NODE_G):
        pltpu.async_copy(ntab.at[nidx_v.at[pl.ds(j * NODE_G, NODE_G)]],
                         nemb_v.at[pl.ds(j * NODE_G, NODE_G)], sem).wait()
    pltpu.sync_copy(x.at[pl.ds(nbase, NODE_PW)], nx_v)
    pltpu.sync_copy(nemb_v, xcat.at[pl.ds(nbase, NODE_PW), pl.ds(0, NTYPE_DIM)])
    pltpu.sync_copy(nx_v, xcat.at[pl.ds(nbase, NODE_PW), pl.ds(NTYPE_DIM, D_FEAT)])

    @pl.when(wid == 0)
    def _node_tail():
        tbase = NW * NODE_PW
        pltpu.sync_copy(ntypes.at[pl.ds(tbase, NODE_TAIL)],
                        nidx_v.at[pl.ds(0, NODE_TAIL)])
        pltpu.async_copy(ntab.at[nidx_v.at[pl.ds(0, NODE_TAIL)]],
                         nemb_v.at[pl.ds(0, NODE_TAIL)], sem).wait()
        pltpu.sync_copy(x.at[pl.ds(tbase, NODE_TAIL)],
                        nx_v.at[pl.ds(0, NODE_TAIL)])
        pltpu.sync_copy(nemb_v.at[pl.ds(0, NODE_TAIL)],
                        xcat.at[pl.ds(tbase, NODE_TAIL), pl.ds(0, NTYPE_DIM)])
        pltpu.sync_copy(nx_v.at[pl.ds(0, NODE_TAIL)],
                        xcat.at[pl.ds(tbase, NODE_TAIL), pl.ds(NTYPE_DIM, D_FEAT)])

    ebase = wid * EDGE_PW

    def edge_chunk(k, carry):
        base = pl.multiple_of(ebase + k * EC, 8)
        pltpu.sync_copy(etypes.at[pl.ds(base, EC)], eidx_v)
        pltpu.sync_copy(eattr.at[pl.ds(base, EC)], efeat_v)
        for j in range(EC // EG):
            pltpu.async_copy(etab.at[eidx_v.at[pl.ds(j * EG, EG)]],
                             eemb_v.at[pl.ds(j * EG, EG)], sem).wait()
        pltpu.sync_copy(eemb_v, ecat.at[pl.ds(base, EC), pl.ds(0, ETYPE_DIM)])
        pltpu.sync_copy(efeat_v, ecat.at[pl.ds(base, EC), pl.ds(ETYPE_DIM, D_EDGE)])
        return carry

    lax.fori_loop(0, NB, edge_chunk, 0)


@functools.partial(jax.jit, static_argnames=())
def kernel(x, eattr, ntypes, etypes, ntype_table, etype_table):
    run = pl.kernel(
        _body,
        out_type=(
            jax.ShapeDtypeStruct((N, NTYPE_DIM + D_FEAT), jnp.float32),
            jax.ShapeDtypeStruct((E, ETYPE_DIM + D_EDGE), jnp.float32),
        ),
        mesh=plsc.VectorSubcoreMesh(core_axis_name="c", subcore_axis_name="s"),
        compiler_params=pltpu.CompilerParams(use_tc_tiling_on_sc=False),
        scratch_types=[
            pltpu.VMEM((NODE_PW,), jnp.int32),
            pltpu.VMEM((NODE_PW, NTYPE_DIM), jnp.float32),
            pltpu.VMEM((NODE_PW, D_FEAT), jnp.float32),
            pltpu.VMEM((EC,), jnp.int32),
            pltpu.VMEM((EC, ETYPE_DIM), jnp.float32),
            pltpu.VMEM((EC, D_EDGE), jnp.float32),
            pltpu.SemaphoreType.DMA,
        ],
    )
    x_cat, eattr_cat = run(x, eattr, ntypes.astype(jnp.int32),
                           etypes.astype(jnp.int32), ntype_table, etype_table)
    return (x_cat, eattr_cat)
